# Initial kernel scaffold; baseline (speedup 1.0000x reference)
import jax
import jax.numpy as jnp
from jax.experimental import pallas as pl


def kernel(user_indices, item_indices, emb_user_mlp, emb_item_mlp, emb_user_mf, emb_item_mf, W0, b0, W1, b1, W2, b2, Wa, ba):
    raise NotImplementedError("write your pallas kernel here")


# speedup vs baseline: 1.0068x; 1.0068x over previous
import functools

import jax
import jax.numpy as jnp
from jax import lax
from jax.experimental import pallas as pl
from jax.experimental.pallas import tpu as pltpu
from jax.experimental.pallas import tpu_sc as plsc

BATCH = 16384
DIM_MLP = 128
DIM_MF = 64

_NUM_CORES = 2
_NUM_SUBCORES = 16
_NW = _NUM_CORES * _NUM_SUBCORES
_BPW = BATCH // _NW
_CH = 128
_NCHUNK = _BPW // _CH


def _sc_gather_body(uidx_hbm, iidx_hbm, eum_hbm, eim_hbm, euf_hbm, eif_hbm,
                    out_um, out_im, out_uf, out_if,
                    uix_v, iix_v, um_v, im_v, uf_v, if_v, sem):
    wid = lax.axis_index("s") * _NUM_CORES + lax.axis_index("c")
    for g in range(_NCHUNK):
        base = wid * _BPW + g * _CH
        pltpu.sync_copy(uidx_hbm.at[pl.ds(base, _CH)], uix_v)
        pltpu.sync_copy(iidx_hbm.at[pl.ds(base, _CH)], iix_v)
        c0 = pltpu.async_copy(eum_hbm.at[uix_v], um_v, sem)
        c1 = pltpu.async_copy(eim_hbm.at[iix_v], im_v, sem)
        c2 = pltpu.async_copy(euf_hbm.at[uix_v], uf_v, sem)
        c3 = pltpu.async_copy(eif_hbm.at[iix_v], if_v, sem)
        c0.wait()
        c1.wait()
        c2.wait()
        c3.wait()
        pltpu.sync_copy(um_v, out_um.at[pl.ds(base, _CH)])
        pltpu.sync_copy(im_v, out_im.at[pl.ds(base, _CH)])
        pltpu.sync_copy(uf_v, out_uf.at[pl.ds(base, _CH)])
        pltpu.sync_copy(if_v, out_if.at[pl.ds(base, _CH)])


_sc_gather = functools.partial(
    pl.kernel,
    mesh=plsc.VectorSubcoreMesh(core_axis_name="c", subcore_axis_name="s"),
    out_type=(
        jax.ShapeDtypeStruct((BATCH, DIM_MLP), jnp.float32),
        jax.ShapeDtypeStruct((BATCH, DIM_MLP), jnp.float32),
        jax.ShapeDtypeStruct((BATCH, DIM_MF), jnp.float32),
        jax.ShapeDtypeStruct((BATCH, DIM_MF), jnp.float32),
    ),
    scratch_types=[
        pltpu.VMEM((_CH,), jnp.int32),
        pltpu.VMEM((_CH,), jnp.int32),
        pltpu.VMEM((_CH, DIM_MLP), jnp.float32),
        pltpu.VMEM((_CH, DIM_MLP), jnp.float32),
        pltpu.VMEM((_CH, DIM_MF), jnp.float32),
        pltpu.VMEM((_CH, DIM_MF), jnp.float32),
        pltpu.SemaphoreType.DMA,
    ],
    compiler_params=pltpu.CompilerParams(use_tc_tiling_on_sc=False),
)(_sc_gather_body)


_BB = 1024


def _mlp_body(ue_ref, ie_ref, uf_ref, if_ref,
              w0a_ref, w0b_ref, b0_ref, w1_ref, b1_ref, w2_ref, b2_ref,
              wam_ref, waf_ref, ba_ref, out_ref):
    f32 = jnp.float32
    h = jnp.dot(ue_ref[...], w0a_ref[...], preferred_element_type=f32)
    h += jnp.dot(ie_ref[...], w0b_ref[...], preferred_element_type=f32)
    h = jnp.maximum(h + b0_ref[...], 0.0)
    h = jnp.maximum(jnp.dot(h, w1_ref[...], preferred_element_type=f32) + b1_ref[...], 0.0)
    h = jnp.maximum(jnp.dot(h, w2_ref[...], preferred_element_type=f32) + b2_ref[...], 0.0)
    mf = uf_ref[...] * if_ref[...]
    logit = (jnp.dot(h, wam_ref[...], preferred_element_type=f32)
             + jnp.dot(mf, waf_ref[...], preferred_element_type=f32)
             + ba_ref[0, 0])
    out_ref[...] = jax.nn.sigmoid(logit)


def _mlp_call(ue, ie, uf, if_, w0a, w0b, b0, w1, b1, w2, b2, wam, waf, ba):
    grid = BATCH // _BB
    bspec_row = lambda d: pl.BlockSpec((_BB, d), lambda i: (i, 0))
    bspec_full = lambda s: pl.BlockSpec(s, lambda i: (0, 0))
    return pl.pallas_call(
        _mlp_body,
        grid=(grid,),
        in_specs=[
            bspec_row(DIM_MLP), bspec_row(DIM_MLP), bspec_row(DIM_MF), bspec_row(DIM_MF),
            bspec_full((DIM_MLP, 256)), bspec_full((DIM_MLP, 256)), bspec_full((1, 256)),
            bspec_full((256, 128)), bspec_full((1, 128)),
            bspec_full((128, 64)), bspec_full((1, 64)),
            bspec_full((64, 1)), bspec_full((64, 1)), bspec_full((1, 1)),
        ],
        out_specs=pl.BlockSpec((_BB, 1), lambda i: (i, 0)),
        out_shape=jax.ShapeDtypeStruct((BATCH, 1), jnp.float32),
        compiler_params=pltpu.CompilerParams(
            dimension_semantics=("arbitrary",),
        ),
    )(ue, ie, uf, if_, w0a, w0b, b0, w1, b1, w2, b2, wam, waf, ba)


def kernel(user_indices, item_indices, emb_user_mlp, emb_item_mlp,
           emb_user_mf, emb_item_mf, W0, b0, W1, b1, W2, b2, Wa, ba):
    ui = user_indices.astype(jnp.int32)
    ii = item_indices.astype(jnp.int32)
    ue, ie, uf, if_ = _sc_gather(ui, ii, emb_user_mlp, emb_item_mlp,
                                 emb_user_mf, emb_item_mf)
    w0a = W0[:DIM_MLP]
    w0b = W0[DIM_MLP:]
    wam = Wa[:64]
    waf = Wa[64:]
    return _mlp_call(ue, ie, uf, if_, w0a, w0b, b0.reshape(1, -1),
                     W1, b1.reshape(1, -1), W2, b2.reshape(1, -1),
                     wam, waf, ba.reshape(1, 1))
